# baseline (speedup 1.0000x reference)
import functools

import jax
import jax.numpy as jnp
from jax import lax
from jax.experimental import pallas as pl
from jax.experimental.pallas import tpu as pltpu
from jax.experimental.pallas import tpu_sc as plsc

N_ROWS = 16384
N_COLS = 1024
HALF = N_ROWS // 2
K = int(0.05 * HALF)

_NC = 2
_NS = 16
_L = 16

_T_CHUNK = HALF // _NS
_S_CHUNK = N_ROWS // (_NC * _NS)

_ROWS_PER_STEP = 1024

_GATHER_DNUMS = lax.GatherDimensionNumbers(
    offset_dims=(), collapsed_slice_dims=(0,), start_index_map=(0,))


def _lane_shuffle(v, idx):
    return lax.gather(v, idx[:, None], _GATHER_DNUMS, slice_sizes=(1,),
                      mode=lax.GatherScatterMode.PROMISE_IN_BOUNDS)


def _butterfly(v, op):
    idx = lax.iota(jnp.int32, _L)
    for s in (8, 4, 2, 1):
        v = op(v, _lane_shuffle(v, idx ^ s))
    return v


def _lane_sum(v):
    return _butterfly(v, lax.add)


def _lane_max(v):
    return _butterfly(v, jnp.maximum)


def _lane_min(v):
    return _butterfly(v, jnp.minimum)


def _mse_body(o_ref, d_ref, e_ref):
    diff = o_ref[...] - d_ref[...]
    s = jnp.sum(diff * diff, axis=1)
    e_ref[0, 0, :] = s * (1.0 / N_COLS)


def _compute_errors(original, decoded):
    grid = N_ROWS // _ROWS_PER_STEP
    err3 = pl.pallas_call(
        _mse_body,
        grid=(grid,),
        in_specs=[
            pl.BlockSpec((_ROWS_PER_STEP, N_COLS), lambda i: (i, 0)),
            pl.BlockSpec((_ROWS_PER_STEP, N_COLS), lambda i: (i, 0)),
        ],
        out_specs=pl.BlockSpec((1, 1, _ROWS_PER_STEP), lambda i: (i, 0, 0)),
        out_shape=jax.ShapeDtypeStruct((grid, 1, _ROWS_PER_STEP), jnp.float32),
    )(original, decoded)
    return err3.reshape(N_ROWS)


@functools.partial(
    pl.kernel,
    mesh=plsc.VectorSubcoreMesh(core_axis_name="c", subcore_axis_name="s"),
    out_type=(
        jax.ShapeDtypeStruct((N_ROWS,), jnp.float32),
        jax.ShapeDtypeStruct((8, _NC, _NS, _L), jnp.int32),
        jax.ShapeDtypeStruct((_NC, _NS, _L), jnp.float32),
    ),
    scratch_types=[
        pltpu.VMEM((_T_CHUNK,), jnp.int32),
        pltpu.VMEM((_T_CHUNK,), jnp.float32),
        pltpu.VMEM((_L,), jnp.int32),
        pltpu.VMEM((_NS, _L), jnp.int32),
        pltpu.VMEM((_L,), jnp.float32),
        pltpu.VMEM((_NS, _L), jnp.float32),
    ],
)
def _sc_threshold_sigmoid(err_hbm, bits_hbm, out_hbm, xcnt_hbm, xmin_hbm,
                          ints_v, fl_v, cnt_v, all_cnt_v, min_v, all_min_v):
    cid = lax.axis_index("c")
    sid = lax.axis_index("s")
    wid = sid * _NC + cid
    iota = lax.iota(jnp.int32, _L)

    pltpu.sync_copy(bits_hbm.at[pl.ds(sid * _T_CHUNK, _T_CHUNK)], ints_v)
    pltpu.sync_copy(err_hbm.at[pl.ds(sid * _T_CHUNK, _T_CHUNK)], fl_v)

    def count_round(r, t, shift, n_cand):
        cands = []
        for v in range(1, n_cand + 1):
            vs = lax.shift_left(jnp.int32(v), shift)
            cands.append(t | vs)
        accs = [jnp.zeros((_L,), jnp.int32) for _ in range(n_cand)]
        for j in range(_T_CHUNK // _L):
            x = ints_v[pl.ds(j * _L, _L)]
            for v in range(n_cand):
                accs[v] = accs[v] + jnp.where(x >= cands[v], 1, 0)
        packed = jnp.zeros((_L,), jnp.int32)
        for v in range(n_cand):
            packed = jnp.where(iota == (v + 1), _lane_sum(accs[v]), packed)
        cnt_v[...] = packed
        pltpu.sync_copy(cnt_v, xcnt_hbm.at[r, cid, sid])
        plsc.subcore_barrier()
        pltpu.sync_copy(xcnt_hbm.at[r, cid], all_cnt_v)
        g = jnp.zeros((_L,), jnp.int32)
        for row in range(_NS):
            g = g + all_cnt_v[row]
        sel = jnp.where(g >= K, iota, 0)
        vstar = _lane_max(sel)
        return t | lax.shift_left(vstar, shift)

    def round_body(i, t):
        return count_round(i, t, jnp.int32(27) - 4 * i, 15)

    t_bits = lax.fori_loop(0, 7, round_body, jnp.zeros((_L,), jnp.int32))
    t_bits = count_round(7, t_bits, jnp.int32(0), 7)

    m = jnp.full((_L,), jnp.inf, jnp.float32)
    for j in range(_T_CHUNK // _L):
        xi = ints_v[pl.ds(j * _L, _L)]
        xf = fl_v[pl.ds(j * _L, _L)]
        m = jnp.minimum(m, jnp.where(xi >= t_bits, xf, jnp.inf))
    min_v[...] = m
    pltpu.sync_copy(min_v, xmin_hbm.at[cid, sid])
    plsc.subcore_barrier()
    pltpu.sync_copy(xmin_hbm.at[cid], all_min_v)
    mm = jnp.full((_L,), jnp.inf, jnp.float32)
    for row in range(_NS):
        mm = jnp.minimum(mm, all_min_v[row])
    thr = _lane_min(mm)

    pltpu.sync_copy(err_hbm.at[pl.ds(wid * _S_CHUNK, _S_CHUNK)], fl_v)
    for j in range(_S_CHUNK // _L):
        x = fl_v[pl.ds(j * _L, _L)]
        fl_v[pl.ds(j * _L, _L)] = 1.0 / (1.0 + jnp.exp(thr - x))
    pltpu.sync_copy(fl_v, out_hbm.at[pl.ds(wid * _S_CHUNK, _S_CHUNK)])


def kernel(original, decoded, encoded):
    del encoded
    err = _compute_errors(original, decoded)
    bits = lax.bitcast_convert_type(err[:HALF], jnp.int32)
    out, _, _ = _sc_threshold_sigmoid(err, bits)
    return out
